# baseline (speedup 1.0000x reference)
import functools

import jax
import jax.numpy as jnp
from jax import lax
from jax.experimental import pallas as pl
from jax.experimental.pallas import tpu as pltpu
from jax.experimental.pallas import tpu_sc as plsc

N = 10000
E = 320000
D = 128
EPS = 1e-5

NW = 32
R = 320
NPAD = NW * R
SENT = R
CA = 4000
CA2 = CA // 2
NSUB = 2
STRIDE = E // 2 + CA + 512
CB = 256
CBH = CB // 2
DW = D // 2

_SC_PARAMS = pltpu.CompilerParams(needs_layout_passes=False)
_SC_PARAMS_UNTILED = pltpu.CompilerParams(needs_layout_passes=False,
                                          use_tc_tiling_on_sc=False)


def _sc_mesh():
    return plsc.VectorSubcoreMesh(core_axis_name="c", subcore_axis_name="s")


def _worker_id():
    return lax.axis_index("s") * 2 + lax.axis_index("c")


def _bucket_body(src_hbm, dst_hbm, esrc_hbm, edl_hbm, cnt_hbm,
                 sbuf0, sbuf1, dbuf0, dbuf1,
                 csra0, csra1, cdla0, cdla1, csrb0, csrb1, cdlb0, cdlb1,
                 cstage, seml0, seml1, semf):
    w = _worker_id()
    lo = w * R
    nchunk = E // CA
    insets = ((sbuf0, dbuf0, seml0), (sbuf1, dbuf1, seml1))
    outsets = ((((csra0, cdla0), (csrb0, cdlb0))),
               (((csra1, cdla1), (csrb1, cdlb1))))

    def sub_base(t, off):
        return pl.multiple_of((NSUB * w + t) * STRIDE + off, 8)

    def load(c, s):
        sbuf, dbuf, seml = s
        cc = jnp.minimum(c, nchunk - 1)
        base = pl.multiple_of(cc * CA, 8)
        pltpu.async_copy(src_hbm.at[pl.ds(base, CA)], sbuf, seml)
        pltpu.async_copy(dst_hbm.at[pl.ds(base, CA)], dbuf, seml)

    def wait_load(s):
        sbuf, dbuf, seml = s
        pltpu.make_async_copy(src_hbm.at[pl.ds(0, CA)], sbuf, seml).wait()
        pltpu.make_async_copy(dst_hbm.at[pl.ds(0, CA)], dbuf, seml).wait()

    def wait_flush(o):
        for t in range(NSUB):
            csr, cdl = o[t]
            base = sub_base(t, 0)
            pltpu.make_async_copy(csr, esrc_hbm.at[pl.ds(base, CA2 + 16)],
                                  semf).wait()
            pltpu.make_async_copy(cdl, edl_hbm.at[pl.ds(base, CA2 + 16)],
                                  semf).wait()

    def phase(b, si, offs, first):
        ins = insets[si]
        outs = outsets[si]
        load(b + 1, insets[1 - si])
        wait_load(ins)
        sbuf, dbuf, _ = ins

        def grp(g, wps):
            wpa, wpb = wps
            da = dbuf[pl.ds(g * 32, 16)]
            sa = sbuf[pl.ds(g * 32, 16)]
            db = dbuf[pl.ds(g * 32 + 16, 16)]
            sb = sbuf[pl.ds(g * 32 + 16, 16)]
            dla = da - lo
            dlb = db - lo
            ma = (dla >= 0) & (dla < R)
            mb = (dlb >= 0) & (dlb < R)
            plsc.store_compressed(outs[0][1].at[pl.ds(wpa, 16)], dla, mask=ma)
            plsc.store_compressed(outs[0][0].at[pl.ds(wpa, 16)], sa, mask=ma)
            plsc.store_compressed(outs[1][1].at[pl.ds(wpb, 16)], dlb, mask=mb)
            plsc.store_compressed(outs[1][0].at[pl.ds(wpb, 16)], sb, mask=mb)
            na = plsc.all_reduce_population_count(ma)
            nb = plsc.all_reduce_population_count(mb)
            return (wpa + na[0], wpb + nb[0])

        wps = lax.fori_loop(0, CA // 32, grp, (0, 0))
        if not first:
            wait_flush(outsets[1 - si])
        new_offs = []
        for t in range(NSUB):
            csr, cdl = outs[t]
            wp = wps[t]
            cdl[pl.ds(wp, 16)] = jnp.full((16,), SENT, jnp.int32)
            csr[pl.ds(wp, 16)] = jnp.zeros((16,), jnp.int32)
            base = sub_base(t, offs[t])
            pltpu.async_copy(csr, esrc_hbm.at[pl.ds(base, CA2 + 16)], semf)
            pltpu.async_copy(cdl, edl_hbm.at[pl.ds(base, CA2 + 16)], semf)
            new_offs.append(offs[t] + ((wp + 7) & ~7))
        return tuple(new_offs)

    load(0, insets[0])
    offs = phase(0, 0, (0, 0), True)

    def pair(i, offs):
        offs = phase(i * 2 + 1, 1, offs, False)
        offs = phase(i * 2 + 2, 0, offs, False)
        return offs

    offs = lax.fori_loop(0, (nchunk - 2) // 2, pair, offs)
    offs = phase(nchunk - 1, 1, offs, False)
    wait_load(insets[0])
    wait_flush(outsets[1])

    for t in range(NSUB):
        cstage[...] = jnp.full((16,), offs[t], jnp.int32)
        pltpu.sync_copy(
            cstage,
            cnt_hbm.at[pl.ds(pl.multiple_of((NSUB * w + t) * 16, 8), 16)])


def _make_bucket():
    return functools.partial(
        pl.kernel,
        out_type=[
            jax.ShapeDtypeStruct((NW * NSUB * STRIDE,), jnp.int32),
            jax.ShapeDtypeStruct((NW * NSUB * STRIDE,), jnp.int32),
            jax.ShapeDtypeStruct((NW * NSUB * 16,), jnp.int32),
        ],
        mesh=_sc_mesh(),
        scratch_types=[
            pltpu.VMEM((CA,), jnp.int32),
            pltpu.VMEM((CA,), jnp.int32),
            pltpu.VMEM((CA,), jnp.int32),
            pltpu.VMEM((CA,), jnp.int32),
            pltpu.VMEM((CA2 + 16,), jnp.int32),
            pltpu.VMEM((CA2 + 16,), jnp.int32),
            pltpu.VMEM((CA2 + 16,), jnp.int32),
            pltpu.VMEM((CA2 + 16,), jnp.int32),
            pltpu.VMEM((CA2 + 16,), jnp.int32),
            pltpu.VMEM((CA2 + 16,), jnp.int32),
            pltpu.VMEM((CA2 + 16,), jnp.int32),
            pltpu.VMEM((CA2 + 16,), jnp.int32),
            pltpu.VMEM((16,), jnp.int32),
            pltpu.SemaphoreType.DMA,
            pltpu.SemaphoreType.DMA,
            pltpu.SemaphoreType.DMA,
        ],
        compiler_params=_SC_PARAMS,
    )(_bucket_body)


def _segmax_body(m_hbm, esrc_hbm, edl_hbm, cnt_hbm, agg_hbm,
                 sidx0, sidx1, dloc0, dloc1, rows0, rows1, acc0, acc1,
                 cstage, sem0, sem1, seml0, seml1):
    w = _worker_id()
    sets = ((sidx0, dloc0, rows0, sem0, seml0),
            (sidx1, dloc1, rows1, sem1, seml1))
    accs = (acc0, acc1)

    def zrow(i, _):
        for a in accs:
            for j in range(D // 16):
                a[i, pl.ds(j * 16, 16)] = jnp.zeros((16,), jnp.float32)
        return 0
    lax.fori_loop(0, R + 8, zrow, 0)

    def process(t):
        sbase = (NSUB * w + t) * STRIDE
        pltpu.sync_copy(
            cnt_hbm.at[pl.ds(pl.multiple_of((NSUB * w + t) * 16, 8), 16)],
            cstage)
        cnt = cstage[pl.ds(0, 16)][0]
        nb = lax.div(cnt + (CB - 1), CB)

        def lists_start(b, s):
            sidx, dloc, rows, sem, seml = s
            base = pl.multiple_of(sbase + b * CB, 8)
            pltpu.async_copy(esrc_hbm.at[pl.ds(base, CB)], sidx, seml)
            pltpu.async_copy(edl_hbm.at[pl.ds(base, CB)],
                             dloc.at[pl.ds(0, CB)], seml)

        def gather_start(b, s):
            sidx, dloc, rows, sem, seml = s
            base = pl.multiple_of(sbase + b * CB, 8)
            pltpu.make_async_copy(esrc_hbm.at[pl.ds(base, CB)], sidx,
                                  seml).wait()
            pltpu.make_async_copy(edl_hbm.at[pl.ds(base, CB)],
                                  dloc.at[pl.ds(0, CB)], seml).wait()
            for g in range(CB // 16):
                v = sidx[pl.ds(g * 16, 16)]
                sidx[pl.ds(g * 16, 16)] = jnp.clip(v, 0, N - 1)
            pltpu.async_copy(m_hbm.at[sidx.at[pl.ds(0, CBH)]],
                             rows.at[pl.ds(0, CBH)], sem)
            pltpu.async_copy(m_hbm.at[sidx.at[pl.ds(CBH, CBH)]],
                             rows.at[pl.ds(CBH, CBH)], sem)

        def consume(b, s):
            sidx, dloc, rows, sem, seml = s

            def group(g, _):
                e0 = b * CB + g * 16
                dv = dloc[pl.ds(g * 16, 16)]
                pos = jnp.full((16,), e0, jnp.int32) + lax.iota(jnp.int32, 16)
                dv = jnp.where(pos < cnt, dv, SENT)
                dv = jnp.clip(dv, 0, SENT)
                nw_ = DW // 16
                nj = D // 16
                js = [pl.ds(j * 16, 16) for j in range(nj)]

                def rows_of(l):
                    return ([rows[g * 16 + l, js[j]] for j in range(nw_)]
                            + [rows[g * 16 + l + 1, js[j]]
                               for j in range(nw_)])

                def unpacked(rw):
                    lows, highs = [], []
                    for jj in range(nw_):
                        bf = plsc.bitcast(rw[jj], jnp.bfloat16)
                        a, b = plsc.unpack(
                            bf, format=plsc.PackFormat.INTERLEAVED)
                        lows.append(a)
                        highs.append(b)
                    return lows + highs

                rcur = rows_of(0)
                for l in range(0, 16, 2):
                    d0 = dv[l]
                    d1 = dv[l + 1]
                    r0 = unpacked(rcur[:nw_])
                    r1 = unpacked(rcur[nw_:])
                    a0 = [acc0[d0, js[j]] for j in range(nj)]
                    a1 = [acc1[d1, js[j]] for j in range(nj)]
                    rnext = rows_of(l + 2) if l < 14 else None
                    for j in range(nj):
                        acc0[d0, js[j]] = jnp.maximum(a0[j], r0[j])
                    for j in range(nj):
                        acc1[d1, js[j]] = jnp.maximum(a1[j], r1[j])
                    rcur = rnext
                return 0
            lax.fori_loop(0, CB // 16, group, 0)

        def wait_gather(s):
            sidx, dloc, rows, sem, seml = s
            pltpu.make_async_copy(m_hbm.at[sidx.at[pl.ds(0, CBH)]],
                                  rows.at[pl.ds(0, CBH)], sem).wait()
            pltpu.make_async_copy(m_hbm.at[sidx.at[pl.ds(CBH, CBH)]],
                                  rows.at[pl.ds(CBH, CBH)], sem).wait()

        lists_start(0, sets[0])
        gather_start(0, sets[0])
        lists_start(1, sets[1])

        def phase(b, s, so):
            gather_start(b, s)
            wait_gather(so)
            consume(b - 1, so)
            lists_start(b + 1, so)

        def pair(i, _):
            phase(i * 2 + 1, sets[1], sets[0])
            phase(i * 2 + 2, sets[0], sets[1])
            return 0

        npair = lax.div(nb + 1, 2)
        lax.fori_loop(0, npair, pair, 0)
        wait_gather(sets[0])
        base = pl.multiple_of(sbase, 8)
        pltpu.make_async_copy(esrc_hbm.at[pl.ds(base, CB)], sidx1,
                              seml1).wait()
        pltpu.make_async_copy(edl_hbm.at[pl.ds(base, CB)],
                              dloc1.at[pl.ds(0, CB)], seml1).wait()

    for t in range(NSUB):
        process(t)

    def mrow(i, _):
        for j in range(D // 16):
            acc0[i, pl.ds(j * 16, 16)] = jnp.maximum(
                acc0[i, pl.ds(j * 16, 16)], acc1[i, pl.ds(j * 16, 16)])
        return 0
    lax.fori_loop(0, R, mrow, 0)
    pltpu.sync_copy(acc0.at[pl.ds(0, R)], agg_hbm.at[pl.ds(w * R, R)])


def _make_segmax():
    return functools.partial(
        pl.kernel,
        out_type=jax.ShapeDtypeStruct((NPAD, D), jnp.float32),
        mesh=_sc_mesh(),
        scratch_types=[
            pltpu.VMEM((CB,), jnp.int32),
            pltpu.VMEM((CB,), jnp.int32),
            pltpu.VMEM((CB + 16,), jnp.int32),
            pltpu.VMEM((CB + 16,), jnp.int32),
            pltpu.VMEM((CB, DW), jnp.int32),
            pltpu.VMEM((CB, DW), jnp.int32),
            pltpu.VMEM((R + 8, D), jnp.float32),
            pltpu.VMEM((R + 8, D), jnp.float32),
            pltpu.VMEM((16,), jnp.int32),
            pltpu.SemaphoreType.DMA,
            pltpu.SemaphoreType.DMA,
            pltpu.SemaphoreType.DMA,
            pltpu.SemaphoreType.DMA,
        ],
        compiler_params=_SC_PARAMS_UNTILED,
    )(_segmax_body)


def _norm_in(h, s_ref, q_ref, g_ref, b_ref):
    mu = s_ref[...] * (1.0 / N)
    var = q_ref[...] * (1.0 / N) - mu * mu
    scale = g_ref[...] * lax.rsqrt(var + EPS)
    return (h - mu) * scale + b_ref[...]


def _pool_body(use_norm, *refs):
    if use_norm:
        h_ref, w_ref, b_ref, s_ref, q_ref, g_ref, bt_ref, m_ref = refs
        h = _norm_in(h_ref[...], s_ref, q_ref, g_ref, bt_ref)
    else:
        h_ref, w_ref, b_ref, m_ref = refs
        h = h_ref[...]
    mm = lax.dot_general(h, w_ref[...], (((1,), (1,)), ((), ())),
                         preferred_element_type=jnp.float32)
    m = jnp.maximum(mm + b_ref[...], 0.0)
    mb = m.astype(jnp.bfloat16)
    lo = lax.bitcast_convert_type(mb[:, :DW], jnp.uint16).astype(jnp.uint32)
    hi = lax.bitcast_convert_type(mb[:, DW:], jnp.uint16).astype(jnp.uint32)
    m_ref[...] = lax.bitcast_convert_type(lo | (hi << 16), jnp.int32)


def _out_body(use_norm, use_tanh, stats, *refs):
    if use_norm:
        (h_ref, agg_ref, sw_ref, nw_ref, b_ref, s_ref, q_ref, g_ref, bt_ref,
         *outs) = refs
        h = _norm_in(h_ref[...], s_ref, q_ref, g_ref, bt_ref)
    else:
        h_ref, agg_ref, sw_ref, nw_ref, b_ref, *outs = refs
        h = h_ref[...]
    t = (lax.dot_general(h, sw_ref[...], (((1,), (1,)), ((), ())),
                         preferred_element_type=jnp.float32)
         + lax.dot_general(agg_ref[...], nw_ref[...], (((1,), (1,)), ((), ())),
                           preferred_element_type=jnp.float32)
         + b_ref[...])
    if use_tanh:
        t = jnp.tanh(t)
    outs[0][...] = t
    if stats:
        outs[1][...] = jnp.sum(t, axis=0, keepdims=True)
        outs[2][...] = jnp.sum(t * t, axis=0, keepdims=True)


def _pool_call(h, w, b, norm=None):
    use_norm = norm is not None
    args = [h, w, b.reshape(1, D)]
    if use_norm:
        s, q, g, bt = norm
        args += [s, q, g.reshape(1, D), bt.reshape(1, D)]
    return pl.pallas_call(
        functools.partial(_pool_body, use_norm),
        out_shape=jax.ShapeDtypeStruct((N, DW), jnp.int32),
    )(*args)


def _out_call(h, agg, sw, nw, b, norm=None, use_tanh=True, stats=True):
    use_norm = norm is not None
    args = [h, agg, sw, nw, b.reshape(1, D)]
    if use_norm:
        s, q, g, bt = norm
        args += [s, q, g.reshape(1, D), bt.reshape(1, D)]
    out_shape = [jax.ShapeDtypeStruct((N, D), jnp.float32)]
    if stats:
        out_shape += [jax.ShapeDtypeStruct((1, D), jnp.float32),
                      jax.ShapeDtypeStruct((1, D), jnp.float32)]
    res = pl.pallas_call(
        functools.partial(_out_body, use_norm, use_tanh, stats),
        out_shape=out_shape,
    )(*args)
    return res if stats else res[0]


def kernel(x, edge_index,
           c1_pool_W, c1_pool_b, c1_self_W, c1_neigh_W, c1_bias, c1_gamma,
           c1_beta, c2_pool_W, c2_pool_b, c2_self_W, c2_neigh_W, c2_bias,
           c2_gamma, c2_beta, c3_pool_W, c3_pool_b, c3_self_W, c3_neigh_W,
           c3_bias):
    src = edge_index[0]
    dst = edge_index[1]

    esrc, edl, cnt = _make_bucket()(src, dst)
    segmax = _make_segmax()

    m1 = _pool_call(x, c1_pool_W, c1_pool_b)
    agg1 = segmax(m1, esrc, edl, cnt)[:N]
    t1, s1, q1 = _out_call(x, agg1, c1_self_W, c1_neigh_W, c1_bias)

    n1 = (s1, q1, c1_gamma, c1_beta)
    m2 = _pool_call(t1, c2_pool_W, c2_pool_b, norm=n1)
    agg2 = segmax(m2, esrc, edl, cnt)[:N]
    t2, s2, q2 = _out_call(t1, agg2, c2_self_W, c2_neigh_W, c2_bias, norm=n1)

    n2 = (s2, q2, c2_gamma, c2_beta)
    m3 = _pool_call(t2, c3_pool_W, c3_pool_b, norm=n2)
    agg3 = segmax(m3, esrc, edl, cnt)[:N]
    return _out_call(t2, agg3, c3_self_W, c3_neigh_W, c3_bias, norm=n2,
                     use_tanh=False, stats=False)


# speedup vs baseline: 1.1840x; 1.1840x over previous
import functools

import jax
import jax.numpy as jnp
from jax import lax
from jax.experimental import pallas as pl
from jax.experimental.pallas import tpu as pltpu
from jax.experimental.pallas import tpu_sc as plsc

N = 10000
E = 320000
D = 128
EPS = 1e-5

NW = 32
R = 320
NPAD = NW * R
SENT = R
CA = 4000
CA2 = CA // 2
NSUB = 2
STRIDE = E // 2 + CA + 512
CB = 128
DW = D // 2

_SC_PARAMS = pltpu.CompilerParams(needs_layout_passes=False)
_SC_PARAMS_UNTILED = pltpu.CompilerParams(needs_layout_passes=False,
                                          use_tc_tiling_on_sc=False)


def _sc_mesh():
    return plsc.VectorSubcoreMesh(core_axis_name="c", subcore_axis_name="s")


def _worker_id():
    return lax.axis_index("s") * 2 + lax.axis_index("c")


def _bucket_body(src_hbm, dst_hbm, esrc_hbm, edl_hbm, cnt_hbm,
                 sbuf0, sbuf1, dbuf0, dbuf1,
                 csra0, csra1, cdla0, cdla1, csrb0, csrb1, cdlb0, cdlb1,
                 cstage, seml0, seml1, semf):
    w = _worker_id()
    lo = w * R
    nchunk = E // CA
    insets = ((sbuf0, dbuf0, seml0), (sbuf1, dbuf1, seml1))
    outsets = ((((csra0, cdla0), (csrb0, cdlb0))),
               (((csra1, cdla1), (csrb1, cdlb1))))

    def sub_base(t, off):
        return pl.multiple_of((NSUB * w + t) * STRIDE + off, 8)

    def load(c, s):
        sbuf, dbuf, seml = s
        cc = jnp.minimum(c, nchunk - 1)
        base = pl.multiple_of(cc * CA, 8)
        pltpu.async_copy(src_hbm.at[pl.ds(base, CA)], sbuf, seml)
        pltpu.async_copy(dst_hbm.at[pl.ds(base, CA)], dbuf, seml)

    def wait_load(s):
        sbuf, dbuf, seml = s
        pltpu.make_async_copy(src_hbm.at[pl.ds(0, CA)], sbuf, seml).wait()
        pltpu.make_async_copy(dst_hbm.at[pl.ds(0, CA)], dbuf, seml).wait()

    def wait_flush(o):
        for t in range(NSUB):
            csr, cdl = o[t]
            base = sub_base(t, 0)
            pltpu.make_async_copy(csr, esrc_hbm.at[pl.ds(base, CA2 + 16)],
                                  semf).wait()
            pltpu.make_async_copy(cdl, edl_hbm.at[pl.ds(base, CA2 + 16)],
                                  semf).wait()

    def phase(b, si, offs, first):
        ins = insets[si]
        outs = outsets[si]
        load(b + 1, insets[1 - si])
        wait_load(ins)
        sbuf, dbuf, _ = ins

        def grp(g, wps):
            wpa, wpb = wps
            da = dbuf[pl.ds(g * 32, 16)]
            sa = sbuf[pl.ds(g * 32, 16)]
            db = dbuf[pl.ds(g * 32 + 16, 16)]
            sb = sbuf[pl.ds(g * 32 + 16, 16)]
            dla = da - lo
            dlb = db - lo
            ma = (dla >= 0) & (dla < R)
            mb = (dlb >= 0) & (dlb < R)
            plsc.store_compressed(outs[0][1].at[pl.ds(wpa, 16)], dla, mask=ma)
            plsc.store_compressed(outs[0][0].at[pl.ds(wpa, 16)], sa, mask=ma)
            plsc.store_compressed(outs[1][1].at[pl.ds(wpb, 16)], dlb, mask=mb)
            plsc.store_compressed(outs[1][0].at[pl.ds(wpb, 16)], sb, mask=mb)
            na = plsc.all_reduce_population_count(ma)
            nb = plsc.all_reduce_population_count(mb)
            return (wpa + na[0], wpb + nb[0])

        wps = lax.fori_loop(0, CA // 32, grp, (0, 0))
        if not first:
            wait_flush(outsets[1 - si])
        new_offs = []
        for t in range(NSUB):
            csr, cdl = outs[t]
            wp = wps[t]
            cdl[pl.ds(wp, 16)] = jnp.full((16,), SENT, jnp.int32)
            csr[pl.ds(wp, 16)] = jnp.zeros((16,), jnp.int32)
            base = sub_base(t, offs[t])
            pltpu.async_copy(csr, esrc_hbm.at[pl.ds(base, CA2 + 16)], semf)
            pltpu.async_copy(cdl, edl_hbm.at[pl.ds(base, CA2 + 16)], semf)
            new_offs.append(offs[t] + ((wp + 7) & ~7))
        return tuple(new_offs)

    load(0, insets[0])
    offs = phase(0, 0, (0, 0), True)

    def pair(i, offs):
        offs = phase(i * 2 + 1, 1, offs, False)
        offs = phase(i * 2 + 2, 0, offs, False)
        return offs

    offs = lax.fori_loop(0, (nchunk - 2) // 2, pair, offs)
    offs = phase(nchunk - 1, 1, offs, False)
    wait_load(insets[0])
    wait_flush(outsets[1])

    for t in range(NSUB):
        cstage[...] = jnp.full((16,), offs[t], jnp.int32)
        pltpu.sync_copy(
            cstage,
            cnt_hbm.at[pl.ds(pl.multiple_of((NSUB * w + t) * 16, 8), 16)])


def _make_bucket():
    return functools.partial(
        pl.kernel,
        out_type=[
            jax.ShapeDtypeStruct((NW * NSUB * STRIDE,), jnp.int32),
            jax.ShapeDtypeStruct((NW * NSUB * STRIDE,), jnp.int32),
            jax.ShapeDtypeStruct((NW * NSUB * 16,), jnp.int32),
        ],
        mesh=_sc_mesh(),
        scratch_types=[
            pltpu.VMEM((CA,), jnp.int32),
            pltpu.VMEM((CA,), jnp.int32),
            pltpu.VMEM((CA,), jnp.int32),
            pltpu.VMEM((CA,), jnp.int32),
            pltpu.VMEM((CA2 + 16,), jnp.int32),
            pltpu.VMEM((CA2 + 16,), jnp.int32),
            pltpu.VMEM((CA2 + 16,), jnp.int32),
            pltpu.VMEM((CA2 + 16,), jnp.int32),
            pltpu.VMEM((CA2 + 16,), jnp.int32),
            pltpu.VMEM((CA2 + 16,), jnp.int32),
            pltpu.VMEM((CA2 + 16,), jnp.int32),
            pltpu.VMEM((CA2 + 16,), jnp.int32),
            pltpu.VMEM((16,), jnp.int32),
            pltpu.SemaphoreType.DMA,
            pltpu.SemaphoreType.DMA,
            pltpu.SemaphoreType.DMA,
        ],
        compiler_params=_SC_PARAMS,
    )(_bucket_body)


def _segmax_body(m_hbm, esrc_hbm, edl_hbm, cnt_hbm, agg_hbm,
                 sidx0, sidx1, dloc0, dloc1, rows0, rows1, acc0, acc1,
                 cstage, sem0, sem1, seml0, seml1):
    w = _worker_id()
    sets = ((sidx0, dloc0, rows0, sem0, seml0),
            (sidx1, dloc1, rows1, sem1, seml1))
    accs = (acc0, acc1)

    def zrow(i, _):
        for a in accs:
            for j in range(D // 16):
                a[i, pl.ds(j * 16, 16)] = jnp.zeros((16,), jnp.float32)
        return 0
    lax.fori_loop(0, R + 8, zrow, 0)

    def process(t):
        sbase = (NSUB * w + t) * STRIDE
        pltpu.sync_copy(
            cnt_hbm.at[pl.ds(pl.multiple_of((NSUB * w + t) * 16, 8), 16)],
            cstage)
        cnt = cstage[pl.ds(0, 16)][0]
        nb = lax.div(cnt + (CB - 1), CB)

        def lists_start(b, s):
            sidx, dloc, rows, sem, seml = s
            base = pl.multiple_of(sbase + b * CB, 8)
            pltpu.async_copy(esrc_hbm.at[pl.ds(base, CB)], sidx, seml)
            pltpu.async_copy(edl_hbm.at[pl.ds(base, CB)],
                             dloc.at[pl.ds(0, CB)], seml)

        def gather_start(b, s):
            sidx, dloc, rows, sem, seml = s
            base = pl.multiple_of(sbase + b * CB, 8)
            pltpu.make_async_copy(esrc_hbm.at[pl.ds(base, CB)], sidx,
                                  seml).wait()
            pltpu.make_async_copy(edl_hbm.at[pl.ds(base, CB)],
                                  dloc.at[pl.ds(0, CB)], seml).wait()
            for g in range(CB // 16):
                v = sidx[pl.ds(g * 16, 16)]
                sidx[pl.ds(g * 16, 16)] = jnp.clip(v, 0, N - 1)
            pltpu.async_copy(m_hbm.at[sidx], rows, sem)

        def consume(b, s):
            sidx, dloc, rows, sem, seml = s

            def group(g, _):
                e0 = b * CB + g * 16
                dv = dloc[pl.ds(g * 16, 16)]
                pos = jnp.full((16,), e0, jnp.int32) + lax.iota(jnp.int32, 16)
                dv = jnp.where(pos < cnt, dv, SENT)
                dv = jnp.clip(dv, 0, SENT)
                nw_ = DW // 16
                nj = D // 16
                js = [pl.ds(j * 16, 16) for j in range(nj)]

                def rows_of(l):
                    return ([rows[g * 16 + l, js[j]] for j in range(nw_)]
                            + [rows[g * 16 + l + 1, js[j]]
                               for j in range(nw_)])

                def unpacked(rw):
                    lows, highs = [], []
                    for jj in range(nw_):
                        bf = plsc.bitcast(rw[jj], jnp.bfloat16)
                        a, b = plsc.unpack(
                            bf, format=plsc.PackFormat.INTERLEAVED)
                        lows.append(a)
                        highs.append(b)
                    return lows + highs

                rcur = rows_of(0)
                for l in range(0, 16, 2):
                    d0 = dv[l]
                    d1 = dv[l + 1]
                    r0 = unpacked(rcur[:nw_])
                    r1 = unpacked(rcur[nw_:])
                    a0 = [acc0[d0, js[j]] for j in range(nj)]
                    a1 = [acc1[d1, js[j]] for j in range(nj)]
                    rnext = rows_of(l + 2) if l < 14 else None
                    for j in range(nj):
                        acc0[d0, js[j]] = jnp.maximum(a0[j], r0[j])
                    for j in range(nj):
                        acc1[d1, js[j]] = jnp.maximum(a1[j], r1[j])
                    rcur = rnext
                return 0
            lax.fori_loop(0, CB // 16, group, 0)

        def wait_gather(s):
            sidx, dloc, rows, sem, seml = s
            pltpu.make_async_copy(m_hbm.at[sidx], rows, sem).wait()

        lists_start(0, sets[0])
        gather_start(0, sets[0])
        lists_start(1, sets[1])

        def phase(b, s, so):
            gather_start(b, s)
            wait_gather(so)
            consume(b - 1, so)
            lists_start(b + 1, so)

        def pair(i, _):
            phase(i * 2 + 1, sets[1], sets[0])
            phase(i * 2 + 2, sets[0], sets[1])
            return 0

        npair = lax.div(nb + 1, 2)
        lax.fori_loop(0, npair, pair, 0)
        wait_gather(sets[0])
        base = pl.multiple_of(sbase, 8)
        pltpu.make_async_copy(esrc_hbm.at[pl.ds(base, CB)], sidx1,
                              seml1).wait()
        pltpu.make_async_copy(edl_hbm.at[pl.ds(base, CB)],
                              dloc1.at[pl.ds(0, CB)], seml1).wait()

    for t in range(NSUB):
        process(t)

    def mrow(i, _):
        for j in range(D // 16):
            acc0[i, pl.ds(j * 16, 16)] = jnp.maximum(
                acc0[i, pl.ds(j * 16, 16)], acc1[i, pl.ds(j * 16, 16)])
        return 0
    lax.fori_loop(0, R, mrow, 0)
    pltpu.sync_copy(acc0.at[pl.ds(0, R)], agg_hbm.at[pl.ds(w * R, R)])


def _make_segmax():
    return functools.partial(
        pl.kernel,
        out_type=jax.ShapeDtypeStruct((NPAD, D), jnp.float32),
        mesh=_sc_mesh(),
        scratch_types=[
            pltpu.VMEM((CB,), jnp.int32),
            pltpu.VMEM((CB,), jnp.int32),
            pltpu.VMEM((CB + 16,), jnp.int32),
            pltpu.VMEM((CB + 16,), jnp.int32),
            pltpu.VMEM((CB, DW), jnp.int32),
            pltpu.VMEM((CB, DW), jnp.int32),
            pltpu.VMEM((R + 8, D), jnp.float32),
            pltpu.VMEM((R + 8, D), jnp.float32),
            pltpu.VMEM((16,), jnp.int32),
            pltpu.SemaphoreType.DMA,
            pltpu.SemaphoreType.DMA,
            pltpu.SemaphoreType.DMA,
            pltpu.SemaphoreType.DMA,
        ],
        compiler_params=_SC_PARAMS_UNTILED,
    )(_segmax_body)


def _norm_in(h, s_ref, q_ref, g_ref, b_ref):
    mu = s_ref[...] * (1.0 / N)
    var = q_ref[...] * (1.0 / N) - mu * mu
    scale = g_ref[...] * lax.rsqrt(var + EPS)
    return (h - mu) * scale + b_ref[...]


def _pool_body(use_norm, *refs):
    if use_norm:
        h_ref, w_ref, b_ref, s_ref, q_ref, g_ref, bt_ref, m_ref = refs
        h = _norm_in(h_ref[...], s_ref, q_ref, g_ref, bt_ref)
    else:
        h_ref, w_ref, b_ref, m_ref = refs
        h = h_ref[...]
    mm = lax.dot_general(h, w_ref[...], (((1,), (1,)), ((), ())),
                         preferred_element_type=jnp.float32)
    m = jnp.maximum(mm + b_ref[...], 0.0)
    mb = m.astype(jnp.bfloat16)
    lo = lax.bitcast_convert_type(mb[:, :DW], jnp.uint16).astype(jnp.uint32)
    hi = lax.bitcast_convert_type(mb[:, DW:], jnp.uint16).astype(jnp.uint32)
    m_ref[...] = lax.bitcast_convert_type(lo | (hi << 16), jnp.int32)


def _out_body(use_norm, use_tanh, stats, *refs):
    if use_norm:
        (h_ref, agg_ref, sw_ref, nw_ref, b_ref, s_ref, q_ref, g_ref, bt_ref,
         *outs) = refs
        h = _norm_in(h_ref[...], s_ref, q_ref, g_ref, bt_ref)
    else:
        h_ref, agg_ref, sw_ref, nw_ref, b_ref, *outs = refs
        h = h_ref[...]
    t = (lax.dot_general(h, sw_ref[...], (((1,), (1,)), ((), ())),
                         preferred_element_type=jnp.float32)
         + lax.dot_general(agg_ref[...], nw_ref[...], (((1,), (1,)), ((), ())),
                           preferred_element_type=jnp.float32)
         + b_ref[...])
    if use_tanh:
        t = jnp.tanh(t)
    outs[0][...] = t
    if stats:
        outs[1][...] = jnp.sum(t, axis=0, keepdims=True)
        outs[2][...] = jnp.sum(t * t, axis=0, keepdims=True)


def _pool_call(h, w, b, norm=None):
    use_norm = norm is not None
    args = [h, w, b.reshape(1, D)]
    if use_norm:
        s, q, g, bt = norm
        args += [s, q, g.reshape(1, D), bt.reshape(1, D)]
    return pl.pallas_call(
        functools.partial(_pool_body, use_norm),
        out_shape=jax.ShapeDtypeStruct((N, DW), jnp.int32),
    )(*args)


def _out_call(h, agg, sw, nw, b, norm=None, use_tanh=True, stats=True):
    use_norm = norm is not None
    args = [h, agg, sw, nw, b.reshape(1, D)]
    if use_norm:
        s, q, g, bt = norm
        args += [s, q, g.reshape(1, D), bt.reshape(1, D)]
    out_shape = [jax.ShapeDtypeStruct((N, D), jnp.float32)]
    if stats:
        out_shape += [jax.ShapeDtypeStruct((1, D), jnp.float32),
                      jax.ShapeDtypeStruct((1, D), jnp.float32)]
    res = pl.pallas_call(
        functools.partial(_out_body, use_norm, use_tanh, stats),
        out_shape=out_shape,
    )(*args)
    return res if stats else res[0]


def kernel(x, edge_index,
           c1_pool_W, c1_pool_b, c1_self_W, c1_neigh_W, c1_bias, c1_gamma,
           c1_beta, c2_pool_W, c2_pool_b, c2_self_W, c2_neigh_W, c2_bias,
           c2_gamma, c2_beta, c3_pool_W, c3_pool_b, c3_self_W, c3_neigh_W,
           c3_bias):
    src = edge_index[0]
    dst = edge_index[1]

    esrc, edl, cnt = _make_bucket()(src, dst)
    segmax = _make_segmax()

    m1 = _pool_call(x, c1_pool_W, c1_pool_b)
    agg1 = segmax(m1, esrc, edl, cnt)[:N]
    t1, s1, q1 = _out_call(x, agg1, c1_self_W, c1_neigh_W, c1_bias)

    n1 = (s1, q1, c1_gamma, c1_beta)
    m2 = _pool_call(t1, c2_pool_W, c2_pool_b, norm=n1)
    agg2 = segmax(m2, esrc, edl, cnt)[:N]
    t2, s2, q2 = _out_call(t1, agg2, c2_self_W, c2_neigh_W, c2_bias, norm=n1)

    n2 = (s2, q2, c2_gamma, c2_beta)
    m3 = _pool_call(t2, c3_pool_W, c3_pool_b, norm=n2)
    agg3 = segmax(m3, esrc, edl, cnt)[:N]
    return _out_call(t2, agg3, c3_self_W, c3_neigh_W, c3_bias, norm=n2,
                     use_tanh=False, stats=False)
